# Initial kernel scaffold; baseline (speedup 1.0000x reference)
import jax
import jax.numpy as jnp
from jax.experimental import pallas as pl


def kernel(x, edge_index, w, W, b):
    raise NotImplementedError("write your pallas kernel here")


# speedup vs baseline: 4.7398x; 4.7398x over previous
import functools

import jax
import jax.numpy as jnp
from jax import lax
from jax.experimental import pallas as pl
from jax.experimental.pallas import tpu as pltpu
from jax.experimental.pallas import tpu_sc as plsc

N = 10000
E = 320000
D = 128
NC, NS = 2, 16
NW = NC * NS
G = 112
GPC = G // 16
CH = 90
HALVES = ((0, 48), (48, 42))
HB = 48
E_PAD = NW * CH * G
N_PAD = 10240
RPT = N_PAD // NS
ZR = 80


def _linear_body(x_ref, w_ref, b_ref, o_ref):
    o_ref[...] = lax.dot_general(
        x_ref[...], w_ref[...], (((1,), (1,)), ((), ())),
        preferred_element_type=jnp.float32) + b_ref[...]


def _linear(x, W, b):
    blk = 1000
    return pl.pallas_call(
        _linear_body,
        grid=(N // blk,),
        in_specs=[
            pl.BlockSpec((blk, D), lambda i: (i, 0)),
            pl.BlockSpec((D, D), lambda i: (0, 0)),
            pl.BlockSpec((1, D), lambda i: (0, 0)),
        ],
        out_specs=pl.BlockSpec((blk, D), lambda i: (i, 0)),
        out_shape=jax.ShapeDtypeStruct((N, D), jnp.float32),
    )(x, W, b.reshape(1, D))


def _sc_body(h_hbm, src_hbm, dst_hbm, w_hbm, out_hbm,
             acc, src_v, dst_v, w_v, rows_v, sem):
    cid = lax.axis_index("c")
    sid = lax.axis_index("s")
    wid = cid * NS + sid

    zero = jnp.zeros((16,), jnp.float32)

    def zrow(i, c):
        for j in range(D // 16):
            rows_v[i, pl.ds(j * 16, 16)] = zero
        return c

    lax.fori_loop(0, ZR, zrow, 0)
    for r in range(RPT // ZR):
        pltpu.sync_copy(rows_v.at[pl.ds(0, ZR)],
                        acc.at[pl.ds(sid * RPT + r * ZR, ZR)])
    plsc.subcore_barrier()

    def chunk(c, carry):
        pltpu.async_copy(h_hbm.at[src_v.at[c]], rows_v, sem).wait()

        def group(g, cc):
            w16 = w_v[c, pl.ds(g * 16, 16)]
            for k in range(16):
                r = g * 16 + k
                for j in range(D // 16):
                    sl = pl.ds(j * 16, 16)
                    rows_v[r, sl] = rows_v[r, sl] * w16[k]
            return cc

        lax.fori_loop(0, GPC, group, 0)

        pltpu.sync_copy(rows_v, acc.at[dst_v.at[c]], add=True)
        return carry

    for h0, hc in HALVES:
        pltpu.sync_copy(src_hbm.at[wid, pl.ds(h0, hc)],
                        src_v.at[pl.ds(0, hc)])
        pltpu.sync_copy(dst_hbm.at[wid, pl.ds(h0, hc)],
                        dst_v.at[pl.ds(0, hc)])
        pltpu.sync_copy(w_hbm.at[wid, pl.ds(h0, hc)],
                        w_v.at[pl.ds(0, hc)])
        lax.fori_loop(0, hc, chunk, 0)
    plsc.subcore_barrier()

    pltpu.sync_copy(acc.at[pl.ds(sid * RPT, RPT)],
                    out_hbm.at[cid, pl.ds(sid * RPT, RPT)])


_sc_scatter = functools.partial(
    pl.kernel,
    out_type=jax.ShapeDtypeStruct((NC, N_PAD, D), jnp.float32),
    mesh=plsc.VectorSubcoreMesh(core_axis_name="c", subcore_axis_name="s",
                                num_cores=NC, num_subcores=NS),
    scratch_types=[
        pltpu.VMEM_SHARED((N_PAD, D), jnp.float32),
        pltpu.VMEM((HB, G), jnp.int32),
        pltpu.VMEM((HB, G), jnp.int32),
        pltpu.VMEM((HB, G), jnp.float32),
        pltpu.VMEM((G, D), jnp.float32),
        pltpu.SemaphoreType.DMA,
    ],
)(_sc_body)


def _sum_body(p_ref, o_ref):
    o_ref[...] = p_ref[0] + p_ref[1]


def _sum_parts(p):
    blk = RPT
    return pl.pallas_call(
        _sum_body,
        grid=(N_PAD // blk,),
        in_specs=[pl.BlockSpec((NC, blk, D), lambda i: (0, i, 0))],
        out_specs=pl.BlockSpec((blk, D), lambda i: (i, 0)),
        out_shape=jax.ShapeDtypeStruct((N, D), jnp.float32),
    )(p)


def kernel(x, edge_index, w, W, b):
    h = _linear(x, W, b)
    pad = E_PAD - E
    src = jnp.concatenate(
        [edge_index[0], jnp.zeros((pad,), jnp.int32)]).reshape(NW, CH, G)
    dst = jnp.concatenate(
        [edge_index[1], jnp.zeros((pad,), jnp.int32)]).reshape(NW, CH, G)
    w3 = jnp.concatenate(
        [w, jnp.zeros((pad,), jnp.float32)]).reshape(NW, CH, G)
    parts = _sc_scatter(h, src, dst, w3)
    return _sum_parts(parts)
